# Initial kernel scaffold; baseline (speedup 1.0000x reference)
import functools

import jax
import jax.numpy as jnp
from jax import lax
from jax.experimental import pallas as pl
from jax.experimental.pallas import tpu as pltpu
from jax.experimental.pallas import tpu_sc as plsc

B = 256
N = 200
H = 256
W = 256
N_CHANNELS = 2

NPAD = 208
NCHUNK = NPAD // 16
NW = 32
BPW = B // NW
IMG = H * W


def _sc_scatter(vals, idx):
    mesh = plsc.VectorSubcoreMesh(core_axis_name="c", subcore_axis_name="s")

    @functools.partial(
        pl.kernel,
        out_type=jax.ShapeDtypeStruct((B * N_CHANNELS * IMG,), jnp.float32),
        mesh=mesh,
        scratch_types=[
            pltpu.VMEM((IMG,), jnp.float32),
            pltpu.VMEM((BPW * NPAD,), jnp.float32),
            pltpu.VMEM((BPW * NPAD,), jnp.int32),
        ],
    )
    def k(vals_hbm, idx_hbm, out_hbm, img, vv, iv):
        wid = lax.axis_index("s") * 2 + lax.axis_index("c")
        base = wid * BPW

        pltpu.sync_copy(vals_hbm.at[pl.ds(base * NPAD, BPW * NPAD)], vv)
        pltpu.sync_copy(idx_hbm.at[pl.ds(base * NPAD, BPW * NPAD)], iv)

        zero16 = jnp.zeros((16,), jnp.float32)

        def zbody(i, _):
            for u in range(8):
                img[pl.ds(i * 128 + u * 16, 16)] = zero16
            return _

        lax.fori_loop(0, IMG // 128, zbody, None)

        lanes = lax.iota(jnp.int32, 16)

        def batch_body(b, _):
            for c in range(NCHUNK):
                off = b * NPAD + c * 16
                ivec = iv[pl.ds(off, 16)]
                vvec = vv[pl.ds(off, 16)]
                for j in range(16):
                    plsc.store_scatter(img, [ivec], vvec, mask=lanes == j)

            off0 = (base + b) * (N_CHANNELS * IMG)
            pltpu.sync_copy(img, out_hbm.at[pl.ds(off0, IMG)])
            pltpu.sync_copy(img, out_hbm.at[pl.ds(off0 + IMG, IMG)])

            for c in range(NCHUNK):
                off = b * NPAD + c * 16
                ivec = iv[pl.ds(off, 16)]
                plsc.store_scatter(img, [ivec], zero16)
            return _

        lax.fori_loop(0, BPW, batch_body, None)

    return k(vals, idx)


@jax.jit
def kernel(x):
    powers = x[:, :, 0]
    cx = jnp.round(x[:, :, 1]).astype(jnp.int32)
    cy = jnp.round(x[:, :, 2]).astype(jnp.int32)
    idx = cy * W + cx

    pad_v = jnp.broadcast_to(powers[:, N - 1:N], (B, NPAD - N))
    pad_i = jnp.broadcast_to(idx[:, N - 1:N], (B, NPAD - N))
    vals_p = jnp.concatenate([powers, pad_v], axis=1).reshape(-1)
    idx_p = jnp.concatenate([idx, pad_i], axis=1).reshape(-1)

    out = _sc_scatter(vals_p, idx_p)
    return out.reshape(B, N_CHANNELS, H, W)


# speedup vs baseline: 3.0705x; 3.0705x over previous
import functools

import jax
import jax.numpy as jnp
from jax import lax
from jax.experimental import pallas as pl
from jax.experimental.pallas import tpu as pltpu
from jax.experimental.pallas import tpu_sc as plsc

B = 256
N = 200
H = 256
W = 256
N_CHANNELS = 2

NPAD = 208
NCHUNK = NPAD // 16
NW = 32
BPW = B // NW
IMG = H * W


def _sc_scatter(vals, idx):
    mesh = plsc.VectorSubcoreMesh(core_axis_name="c", subcore_axis_name="s")

    @functools.partial(
        pl.kernel,
        out_type=jax.ShapeDtypeStruct((B * N_CHANNELS * IMG,), jnp.float32),
        mesh=mesh,
        compiler_params=pltpu.CompilerParams(needs_layout_passes=False),
        scratch_types=[
            pltpu.VMEM((IMG,), jnp.float32),
            pltpu.VMEM((BPW * NPAD,), jnp.float32),
            pltpu.VMEM((BPW * NPAD,), jnp.int32),
        ],
    )
    def k(vals_hbm, idx_hbm, out_hbm, img, vv, iv):
        wid = lax.axis_index("s") * 2 + lax.axis_index("c")
        base = wid * BPW

        pltpu.sync_copy(vals_hbm.at[pl.ds(base * NPAD, BPW * NPAD)], vv)
        pltpu.sync_copy(idx_hbm.at[pl.ds(base * NPAD, BPW * NPAD)], iv)

        zero16 = jnp.zeros((16,), jnp.float32)

        def zbody(i, _):
            for u in range(8):
                img[pl.ds(i * 128 + u * 16, 16)] = zero16
            return _

        lax.fori_loop(0, IMG // 128, zbody, None)

        lanes = lax.iota(jnp.int32, 16)

        def batch_body(b, _):
            for c in range(NCHUNK):
                off = b * NPAD + c * 16
                ivec = iv[pl.ds(off, 16)]
                vvec = vv[pl.ds(off, 16)]
                for j in range(16):
                    plsc.store_scatter(img, [ivec], vvec, mask=lanes == j)

            off0 = (base + b) * (N_CHANNELS * IMG)
            pltpu.sync_copy(img, out_hbm.at[pl.ds(off0, IMG)])
            pltpu.sync_copy(img, out_hbm.at[pl.ds(off0 + IMG, IMG)])

            for c in range(NCHUNK):
                off = b * NPAD + c * 16
                ivec = iv[pl.ds(off, 16)]
                plsc.store_scatter(img, [ivec], zero16)
            return _

        lax.fori_loop(0, BPW, batch_body, None)

    return k(vals, idx)


@jax.jit
def kernel(x):
    powers = x[:, :, 0]
    cx = jnp.round(x[:, :, 1]).astype(jnp.int32)
    cy = jnp.round(x[:, :, 2]).astype(jnp.int32)
    idx = cy * W + cx

    keys = (jnp.arange(B, dtype=jnp.int32)[:, None] * (N_CHANNELS * IMG)
            + idx).reshape(-1)
    sk, sv = lax.sort((keys, powers.reshape(-1)), dimension=0, num_keys=1,
                      is_stable=False)
    sidx = (sk & (N_CHANNELS * IMG - 1)).reshape(B, N)
    svals = sv.reshape(B, N)

    pad_v = jnp.broadcast_to(svals[:, N - 1:N], (B, NPAD - N))
    pad_i = jnp.broadcast_to(sidx[:, N - 1:N], (B, NPAD - N))
    vals_p = jnp.concatenate([svals, pad_v], axis=1).reshape(-1)
    idx_p = jnp.concatenate([sidx, pad_i], axis=1).reshape(-1)

    out = _sc_scatter(vals_p, idx_p)
    return out.reshape(B, N_CHANNELS, H, W)


# speedup vs baseline: 7.5417x; 2.4562x over previous
import functools

import jax
import jax.numpy as jnp
from jax import lax
from jax.experimental import pallas as pl
from jax.experimental.pallas import tpu as pltpu
from jax.experimental.pallas import tpu_sc as plsc

B = 256
N = 200
H = 256
W = 256
N_CHANNELS = 2

NPAD = 208
NCHUNK = NPAD // 16
NW = 32
BPW = B // NW
IMG = H * W


def _sc_scatter(vals, idx):
    mesh = plsc.VectorSubcoreMesh(core_axis_name="c", subcore_axis_name="s")

    @functools.partial(
        pl.kernel,
        out_type=jax.ShapeDtypeStruct((B * N_CHANNELS, H, W), jnp.float32),
        mesh=mesh,
        compiler_params=pltpu.CompilerParams(needs_layout_passes=False),
        scratch_types=[
            pltpu.VMEM((H, W), jnp.float32),
            pltpu.VMEM((BPW * NPAD,), jnp.float32),
            pltpu.VMEM((BPW * NPAD,), jnp.int32),
        ],
    )
    def k(vals_hbm, idx_hbm, out_hbm, img, vv, iv):
        wid = lax.axis_index("s") * 2 + lax.axis_index("c")
        base = wid * BPW

        pltpu.sync_copy(vals_hbm.at[pl.ds(base * NPAD, BPW * NPAD)], vv)
        pltpu.sync_copy(idx_hbm.at[pl.ds(base * NPAD, BPW * NPAD)], iv)

        zero16 = jnp.zeros((16,), jnp.float32)

        def zbody(r, _):
            for u in range(W // 16):
                img[r, pl.ds(u * 16, 16)] = zero16
            return _

        lax.fori_loop(0, H, zbody, None)

        lanes = lax.iota(jnp.int32, 16)

        def batch_body(b, _):
            for c in range(NCHUNK):
                off = b * NPAD + c * 16
                ivec = iv[pl.ds(off, 16)]
                vvec = vv[pl.ds(off, 16)]
                iy = lax.shift_right_logical(ivec, 8)
                ix = lax.bitwise_and(ivec, 255)
                for j in range(16):
                    plsc.store_scatter(img, [iy, ix], vvec, mask=lanes == j)

            bc = (base + b) * N_CHANNELS
            pltpu.sync_copy(img, out_hbm.at[bc])
            pltpu.sync_copy(img, out_hbm.at[bc + 1])

            for c in range(NCHUNK):
                off = b * NPAD + c * 16
                ivec = iv[pl.ds(off, 16)]
                iy = lax.shift_right_logical(ivec, 8)
                ix = lax.bitwise_and(ivec, 255)
                plsc.store_scatter(img, [iy, ix], zero16)
            return _

        lax.fori_loop(0, BPW, batch_body, None)

    return k(vals, idx)


@jax.jit
def kernel(x):
    powers = x[:, :, 0]
    cx = jnp.round(x[:, :, 1]).astype(jnp.int32)
    cy = jnp.round(x[:, :, 2]).astype(jnp.int32)
    idx = cy * W + cx

    keys = (jnp.arange(B, dtype=jnp.int32)[:, None] * (N_CHANNELS * IMG)
            + idx).reshape(-1)
    sk, sv = lax.sort((keys, powers.reshape(-1)), dimension=0, num_keys=1,
                      is_stable=False)
    spix = sk & (N_CHANNELS * IMG - 1)
    sidx = spix.reshape(B, N)
    svals = sv.reshape(B, N)

    pad_v = jnp.broadcast_to(svals[:, N - 1:N], (B, NPAD - N))
    pad_i = jnp.broadcast_to(sidx[:, N - 1:N], (B, NPAD - N))
    vals_p = jnp.concatenate([svals, pad_v], axis=1).reshape(-1)
    idx_p = jnp.concatenate([sidx, pad_i], axis=1).reshape(-1)

    out = _sc_scatter(vals_p, idx_p)
    return out.reshape(B, N_CHANNELS, H, W)


# speedup vs baseline: 7.7299x; 1.0249x over previous
import functools

import jax
import jax.numpy as jnp
from jax import lax
from jax.experimental import pallas as pl
from jax.experimental.pallas import tpu as pltpu
from jax.experimental.pallas import tpu_sc as plsc

B = 256
N = 200
H = 256
W = 256
N_CHANNELS = 2

NFULL = N // 16
NTAIL = N - NFULL * 16
NW = 32
BPW = B // NW
IMG = H * W


def _sc_scatter(vals, keys):
    mesh = plsc.VectorSubcoreMesh(core_axis_name="c", subcore_axis_name="s")

    @functools.partial(
        pl.kernel,
        out_type=jax.ShapeDtypeStruct((B * N_CHANNELS, H, W), jnp.float32),
        mesh=mesh,
        compiler_params=pltpu.CompilerParams(needs_layout_passes=False),
        scratch_types=[
            pltpu.VMEM((H, W), jnp.float32),
            pltpu.VMEM((BPW * N,), jnp.float32),
            pltpu.VMEM((BPW * N,), jnp.int32),
        ],
    )
    def k(vals_hbm, keys_hbm, out_hbm, img, vv, iv):
        wid = lax.axis_index("s") * 2 + lax.axis_index("c")
        base = wid * BPW

        pltpu.sync_copy(vals_hbm.at[pl.ds(base * N, BPW * N)], vv)
        pltpu.sync_copy(keys_hbm.at[pl.ds(base * N, BPW * N)], iv)

        zero16 = jnp.zeros((16,), jnp.float32)

        def zbody(r, _):
            for u in range(W // 16):
                img[r, pl.ds(u * 16, 16)] = zero16
            return _

        lax.fori_loop(0, H, zbody, None)

        lanes = lax.iota(jnp.int32, 16)
        tail_mask = lanes < NTAIL

        def batch_body(b, _):
            for c in range(NFULL + 1):
                off = b * N + c * 16
                nlanes = 16 if c < NFULL else NTAIL
                if c == NFULL:
                    off = b * N + N - 16
                ivec = lax.bitwise_and(iv[pl.ds(off, 16)], IMG * N_CHANNELS - 1)
                vvec = vv[pl.ds(off, 16)]
                iy = lax.shift_right_logical(ivec, 8)
                ix = lax.bitwise_and(ivec, 255)
                for j in range(16 - nlanes, 16):
                    plsc.store_scatter(img, [iy, ix], vvec, mask=lanes == j)

            bc = (base + b) * N_CHANNELS
            pltpu.sync_copy(img, out_hbm.at[bc])
            pltpu.sync_copy(img, out_hbm.at[bc + 1])

            for c in range(NFULL + 1):
                off = b * N + (c * 16 if c < NFULL else N - 16)
                ivec = lax.bitwise_and(iv[pl.ds(off, 16)], IMG * N_CHANNELS - 1)
                iy = lax.shift_right_logical(ivec, 8)
                ix = lax.bitwise_and(ivec, 255)
                plsc.store_scatter(img, [iy, ix], zero16)
            return _

        lax.fori_loop(0, BPW, batch_body, None)

    return k(vals, keys)


@jax.jit
def kernel(x):
    powers = x[:, :, 0]
    cx = jnp.round(x[:, :, 1]).astype(jnp.int32)
    cy = jnp.round(x[:, :, 2]).astype(jnp.int32)
    idx = cy * W + cx

    keys = (jnp.arange(B, dtype=jnp.int32)[:, None] * (N_CHANNELS * IMG)
            + idx).reshape(-1)
    sk, sv = lax.sort((keys, powers.reshape(-1)), dimension=0, num_keys=1,
                      is_stable=False)

    out = _sc_scatter(sv, sk)
    return out.reshape(B, N_CHANNELS, H, W)


# speedup vs baseline: 7.7458x; 1.0021x over previous
import functools

import jax
import jax.numpy as jnp
from jax import lax
from jax.experimental import pallas as pl
from jax.experimental.pallas import tpu as pltpu
from jax.experimental.pallas import tpu_sc as plsc

B = 256
N = 200
H = 256
W = 256
N_CHANNELS = 2

NFULL = N // 16
NTAIL = N - NFULL * 16
NW = 32
BPW = B // NW
IMG = H * W


def _sc_scatter(vals, keys):
    mesh = plsc.VectorSubcoreMesh(core_axis_name="c", subcore_axis_name="s")

    @functools.partial(
        pl.kernel,
        out_type=jax.ShapeDtypeStruct((B * N_CHANNELS, H, W), jnp.float32),
        mesh=mesh,
        compiler_params=pltpu.CompilerParams(needs_layout_passes=False),
        scratch_types=[
            pltpu.VMEM((H, W), jnp.float32),
            pltpu.VMEM((BPW * N,), jnp.float32),
            pltpu.VMEM((BPW * N,), jnp.int32),
            pltpu.SemaphoreType.DMA,
        ],
    )
    def k(vals_hbm, keys_hbm, out_hbm, img, vv, iv, sem):
        wid = lax.axis_index("s") * 2 + lax.axis_index("c")
        base = wid * BPW

        pltpu.sync_copy(vals_hbm.at[pl.ds(base * N, BPW * N)], vv)
        pltpu.sync_copy(keys_hbm.at[pl.ds(base * N, BPW * N)], iv)

        zero16 = jnp.zeros((16,), jnp.float32)

        def zbody(r, _):
            for u in range(W // 16):
                img[r, pl.ds(u * 16, 16)] = zero16
            return _

        lax.fori_loop(0, H, zbody, None)

        lanes = lax.iota(jnp.int32, 16)
        tail_mask = lanes < NTAIL

        def batch_body(b, _):
            for c in range(NFULL + 1):
                off = b * N + c * 16
                nlanes = 16 if c < NFULL else NTAIL
                if c == NFULL:
                    off = b * N + N - 16
                ivec = lax.bitwise_and(iv[pl.ds(off, 16)], IMG * N_CHANNELS - 1)
                vvec = vv[pl.ds(off, 16)]
                iy = lax.shift_right_logical(ivec, 8)
                ix = lax.bitwise_and(ivec, 255)
                for j in range(16 - nlanes, 16):
                    plsc.store_scatter(img, [iy, ix], vvec, mask=lanes == j)

            bc = (base + b) * N_CHANNELS
            d0 = pltpu.async_copy(img, out_hbm.at[bc], sem)
            d1 = pltpu.async_copy(img, out_hbm.at[bc + 1], sem)
            d0.wait()
            d1.wait()

            for c in range(NFULL + 1):
                off = b * N + (c * 16 if c < NFULL else N - 16)
                ivec = lax.bitwise_and(iv[pl.ds(off, 16)], IMG * N_CHANNELS - 1)
                iy = lax.shift_right_logical(ivec, 8)
                ix = lax.bitwise_and(ivec, 255)
                plsc.store_scatter(img, [iy, ix], zero16)
            return _

        lax.fori_loop(0, BPW, batch_body, None)

    return k(vals, keys)


@jax.jit
def kernel(x):
    powers = x[:, :, 0]
    cx = jnp.round(x[:, :, 1]).astype(jnp.int32)
    cy = jnp.round(x[:, :, 2]).astype(jnp.int32)
    idx = cy * W + cx

    keys = (jnp.arange(B, dtype=jnp.int32)[:, None] * (N_CHANNELS * IMG)
            + idx).reshape(-1)
    sk, sv = lax.sort((keys, powers.reshape(-1)), dimension=0, num_keys=1,
                      is_stable=False)

    out = _sc_scatter(sv, sk)
    return out.reshape(B, N_CHANNELS, H, W)
